# Initial kernel scaffold; baseline (speedup 1.0000x reference)
import jax
import jax.numpy as jnp
from jax.experimental import pallas as pl


def kernel(edge_index, edge_attr, x, h, msg_W0, msg_b0, msg_W1, msg_b1, msg_W2, msg_b2, vec_W0, vec_b0, vec_W1, vec_b1, vec_W2, vec_b2, sca_W0, sca_b0, sca_W1, sca_b1, sca_W2, sca_b2, nod_W0, nod_b0, nod_W1, nod_b1, nod_W2, nod_b2):
    raise NotImplementedError("write your pallas kernel here")


# speedup vs baseline: 1.0561x; 1.0561x over previous
import functools

import jax
import jax.numpy as jnp
from jax.experimental import pallas as pl
from jax.experimental.pallas import tpu as pltpu

E_BLOCK = 3200
N_BLOCK = 2000


def _silu(v):
    return v * jax.nn.sigmoid(v)


def _edge_body(xij_ref, rest_ref,
               w0x_ref, w0r_ref, b0_ref, w1_ref, b1_ref, w2_ref, b2_ref,
               vw0_ref, vb0_ref, vw1_ref, vb1_ref, vw2_ref, vb2_ref,
               sw0_ref, sb0_ref, sw1_ref, sb1_ref, sw2_ref, sb2_ref,
               vm_ref, sm_ref):
    xij = xij_ref[...]
    n = jnp.sqrt(jnp.sum(xij * xij, axis=1, keepdims=True))
    xn = xij / jnp.maximum(n, 1e-12)
    pre = (xn @ w0x_ref[...] + rest_ref[...] @ w0r_ref[...]) + b0_ref[...]
    l1 = _silu(pre)
    l2 = _silu(l1 @ w1_ref[...] + b1_ref[...])
    msg = l2 @ w2_ref[...] + b2_ref[...]
    v = _silu(msg @ vw0_ref[...] + vb0_ref[...])
    v = _silu(v @ vw1_ref[...] + vb1_ref[...])
    vm_ref[...] = v @ vw2_ref[...] + vb2_ref[...]
    s = _silu(msg @ sw0_ref[...] + sb0_ref[...])
    s = _silu(s @ sw1_ref[...] + sb1_ref[...])
    sm_ref[...] = s @ sw2_ref[...] + sb2_ref[...]


def _node_body(agg_ref, h_ref, vs_ref, x_ref,
               w0_ref, b0_ref, w1_ref, b1_ref, w2_ref, b2_ref,
               vec_ref, na_ref):
    a = _silu(agg_ref[...] @ w0_ref[...] + b0_ref[...])
    a = _silu(a @ w1_ref[...] + b1_ref[...])
    na_ref[...] = (a @ w2_ref[...] + b2_ref[...]) + h_ref[...]
    vec_ref[...] = x_ref[...] + vs_ref[...]


def _full(shape):
    return pl.BlockSpec(shape, lambda i: (0,) * len(shape))


def kernel(edge_index, edge_attr, x, h,
           msg_W0, msg_b0, msg_W1, msg_b1, msg_W2, msg_b2,
           vec_W0, vec_b0, vec_W1, vec_b1, vec_W2, vec_b2,
           sca_W0, sca_b0, sca_W1, sca_b1, sca_W2, sca_b2,
           nod_W0, nod_b0, nod_W1, nod_b1, nod_W2, nod_b2):
    E = edge_index.shape[1]
    N = x.shape[0]
    row = edge_index[0]
    col = edge_index[1]

    xij = jnp.concatenate([x[row], x[col]], axis=1)
    rest = jnp.concatenate([h[row], h[col], edge_attr], axis=1)

    w0x = msg_W0[:6]
    w0r = msg_W0[6:]
    b2 = lambda b: b.reshape(1, -1)

    grid_e = E // E_BLOCK
    vm, sm = pl.pallas_call(
        _edge_body,
        grid=(grid_e,),
        in_specs=[
            pl.BlockSpec((E_BLOCK, 6), lambda i: (i, 0)),
            pl.BlockSpec((E_BLOCK, 80), lambda i: (i, 0)),
            _full(w0x.shape), _full(w0r.shape), _full((1, 64)), _full(msg_W1.shape), _full((1, 64)),
            _full(msg_W2.shape), _full((1, 64)),
            _full(vec_W0.shape), _full((1, 64)), _full(vec_W1.shape), _full((1, 64)),
            _full(vec_W2.shape), _full((1, 3)),
            _full(sca_W0.shape), _full((1, 64)), _full(sca_W1.shape), _full((1, 64)),
            _full(sca_W2.shape), _full((1, 32)),
        ],
        out_specs=[
            pl.BlockSpec((E_BLOCK, 3), lambda i: (i, 0)),
            pl.BlockSpec((E_BLOCK, 32), lambda i: (i, 0)),
        ],
        out_shape=[
            jax.ShapeDtypeStruct((E, 3), jnp.float32),
            jax.ShapeDtypeStruct((E, 32), jnp.float32),
        ],
        compiler_params=pltpu.CompilerParams(
            dimension_semantics=("arbitrary",),
        ),
    )(xij, rest,
      w0x, w0r, b2(msg_b0), msg_W1, b2(msg_b1), msg_W2, b2(msg_b2),
      vec_W0, b2(vec_b0), vec_W1, b2(vec_b1), vec_W2, b2(vec_b2),
      sca_W0, b2(sca_b0), sca_W1, b2(sca_b1), sca_W2, b2(sca_b2))

    vecsum = jax.ops.segment_sum(vm, col, num_segments=N)
    agg = jax.ops.segment_sum(sm, col, num_segments=N)

    grid_n = N // N_BLOCK
    vector, node_attr = pl.pallas_call(
        _node_body,
        grid=(grid_n,),
        in_specs=[
            pl.BlockSpec((N_BLOCK, 32), lambda i: (i, 0)),
            pl.BlockSpec((N_BLOCK, 32), lambda i: (i, 0)),
            pl.BlockSpec((N_BLOCK, 3), lambda i: (i, 0)),
            pl.BlockSpec((N_BLOCK, 3), lambda i: (i, 0)),
            _full(nod_W0.shape), _full((1, 64)), _full(nod_W1.shape), _full((1, 64)),
            _full(nod_W2.shape), _full((1, 32)),
        ],
        out_specs=[
            pl.BlockSpec((N_BLOCK, 3), lambda i: (i, 0)),
            pl.BlockSpec((N_BLOCK, 32), lambda i: (i, 0)),
        ],
        out_shape=[
            jax.ShapeDtypeStruct((N, 3), jnp.float32),
            jax.ShapeDtypeStruct((N, 32), jnp.float32),
        ],
        compiler_params=pltpu.CompilerParams(
            dimension_semantics=("arbitrary",),
        ),
    )(agg, h, vecsum, x,
      nod_W0, b2(nod_b0), nod_W1, b2(nod_b1), nod_W2, b2(nod_b2))

    return vector, node_attr


# speedup vs baseline: 2.5634x; 2.4272x over previous
import functools

import jax
import jax.numpy as jnp
from jax import lax
from jax.experimental import pallas as pl
from jax.experimental.pallas import tpu as pltpu
from jax.experimental.pallas import tpu_sc as plsc

E_BLOCK = 3200
N_BLOCK = 2000
TD = 48
GCH = 1000
N_WORKERS = 32


def _silu(v):
    return v * jax.nn.sigmoid(v)


def _make_gather(E):
    per_w = E // N_WORKERS
    nch = per_w // GCH
    mesh = plsc.VectorSubcoreMesh(core_axis_name="c", subcore_axis_name="s")

    @functools.partial(
        pl.kernel,
        out_type=jax.ShapeDtypeStruct((E, 128), jnp.float32),
        mesh=mesh,
        scratch_types=[
            pltpu.VMEM((GCH,), jnp.int32),
            pltpu.VMEM((GCH,), jnp.int32),
            pltpu.VMEM((GCH, TD), jnp.float32),
            pltpu.VMEM((GCH, TD), jnp.float32),
            pltpu.VMEM((GCH, 16), jnp.float32),
            pltpu.SemaphoreType.DMA,
            pltpu.SemaphoreType.DMA,
            pltpu.SemaphoreType.DMA,
        ],
        compiler_params=pltpu.CompilerParams(use_tc_tiling_on_sc=False),
    )
    def gather_k(t_hbm, row_hbm, col_hbm, ea_hbm, out_hbm,
                 idxr_v, idxc_v, bufr_v, bufc_v, bufe_v, semr, semc, seme):
        wid = lax.axis_index("s") * 2 + lax.axis_index("c")
        base = wid * per_w

        def body(i, carry):
            off = base + i * GCH
            pltpu.sync_copy(row_hbm.at[pl.ds(off, GCH)], idxr_v)
            pltpu.sync_copy(col_hbm.at[pl.ds(off, GCH)], idxc_v)
            ce = pltpu.async_copy(ea_hbm.at[pl.ds(off, GCH)], bufe_v, seme)
            cr = pltpu.async_copy(t_hbm.at[idxr_v], bufr_v, semr)
            cc = pltpu.async_copy(t_hbm.at[idxc_v], bufc_v, semc)
            cr.wait()
            pltpu.sync_copy(bufr_v, out_hbm.at[pl.ds(off, GCH), pl.ds(0, TD)])
            cc.wait()
            pltpu.sync_copy(bufc_v, out_hbm.at[pl.ds(off, GCH), pl.ds(TD, TD)])
            ce.wait()
            pltpu.sync_copy(bufe_v, out_hbm.at[pl.ds(off, GCH), pl.ds(2 * TD, 16)])
            return carry

        lax.fori_loop(0, nch, body, 0)

    return gather_k


def _edge_body(in_ref, xmask_ref,
               w0_ref, b0_ref, w1_ref, b1_ref, w2_ref, b2_ref,
               vw0_ref, vb0_ref, vw1_ref, vb1_ref, vw2_ref, vb2_ref,
               sw0_ref, sb0_ref, sw1_ref, sb1_ref, sw2_ref, sb2_ref,
               vm_ref, sm_ref):
    g = in_ref[...]
    xm = xmask_ref[...]
    n2 = jnp.sum(g * g * xm, axis=1, keepdims=True)
    inv = 1.0 / jnp.maximum(jnp.sqrt(n2), 1e-12)
    scale = xm * inv + (1.0 - xm)
    pre = (g * scale) @ w0_ref[...] + b0_ref[...]
    l1 = _silu(pre)
    l2 = _silu(l1 @ w1_ref[...] + b1_ref[...])
    msg = l2 @ w2_ref[...] + b2_ref[...]
    v = _silu(msg @ vw0_ref[...] + vb0_ref[...])
    v = _silu(v @ vw1_ref[...] + vb1_ref[...])
    vm_ref[...] = v @ vw2_ref[...] + vb2_ref[...]
    s = _silu(msg @ sw0_ref[...] + sb0_ref[...])
    s = _silu(s @ sw1_ref[...] + sb1_ref[...])
    sm_ref[...] = s @ sw2_ref[...] + sb2_ref[...]


def _node_body(agg_ref, h_ref, vs_ref, x_ref,
               w0_ref, b0_ref, w1_ref, b1_ref, w2_ref, b2_ref,
               vec_ref, na_ref):
    a = _silu(agg_ref[...] @ w0_ref[...] + b0_ref[...])
    a = _silu(a @ w1_ref[...] + b1_ref[...])
    na_ref[...] = (a @ w2_ref[...] + b2_ref[...]) + h_ref[...]
    vec_ref[...] = x_ref[...] + vs_ref[...]


def _full(shape):
    return pl.BlockSpec(shape, lambda i: (0,) * len(shape))


def kernel(edge_index, edge_attr, x, h,
           msg_W0, msg_b0, msg_W1, msg_b1, msg_W2, msg_b2,
           vec_W0, vec_b0, vec_W1, vec_b1, vec_W2, vec_b2,
           sca_W0, sca_b0, sca_W1, sca_b1, sca_W2, sca_b2,
           nod_W0, nod_b0, nod_W1, nod_b1, nod_W2, nod_b2):
    E = edge_index.shape[1]
    N = x.shape[0]
    row = edge_index[0]
    col = edge_index[1]

    T = jnp.concatenate(
        [x, jnp.zeros((N, 5), jnp.float32), h, jnp.zeros((N, 8), jnp.float32)],
        axis=1)

    packed = _make_gather(E)(T, row, col, edge_attr)

    z = jnp.zeros((5, 64), jnp.float32)
    z8 = jnp.zeros((8, 64), jnp.float32)
    w0p = jnp.concatenate([
        msg_W0[0:3], z, msg_W0[6:38], z8,
        msg_W0[3:6], z, msg_W0[38:70], z8,
        msg_W0[70:86], jnp.zeros((16, 64), jnp.float32),
    ], axis=0)
    xmask = jnp.zeros((1, 128), jnp.float32)
    xmask = xmask.at[0, 0:3].set(1.0).at[0, 48:51].set(1.0)
    b2 = lambda b: b.reshape(1, -1)

    grid_e = E // E_BLOCK
    vm, sm = pl.pallas_call(
        _edge_body,
        grid=(grid_e,),
        in_specs=[
            pl.BlockSpec((E_BLOCK, 128), lambda i: (i, 0)),
            _full((1, 128)),
            _full((128, 64)), _full((1, 64)),
            _full(msg_W1.shape), _full((1, 64)), _full(msg_W2.shape), _full((1, 64)),
            _full(vec_W0.shape), _full((1, 64)), _full(vec_W1.shape), _full((1, 64)),
            _full(vec_W2.shape), _full((1, 3)),
            _full(sca_W0.shape), _full((1, 64)), _full(sca_W1.shape), _full((1, 64)),
            _full(sca_W2.shape), _full((1, 32)),
        ],
        out_specs=[
            pl.BlockSpec((E_BLOCK, 3), lambda i: (i, 0)),
            pl.BlockSpec((E_BLOCK, 32), lambda i: (i, 0)),
        ],
        out_shape=[
            jax.ShapeDtypeStruct((E, 3), jnp.float32),
            jax.ShapeDtypeStruct((E, 32), jnp.float32),
        ],
        compiler_params=pltpu.CompilerParams(
            dimension_semantics=("arbitrary",),
        ),
    )(packed, xmask,
      w0p, b2(msg_b0), msg_W1, b2(msg_b1), msg_W2, b2(msg_b2),
      vec_W0, b2(vec_b0), vec_W1, b2(vec_b1), vec_W2, b2(vec_b2),
      sca_W0, b2(sca_b0), sca_W1, b2(sca_b1), sca_W2, b2(sca_b2))

    vecsum = jax.ops.segment_sum(vm, col, num_segments=N)
    agg = jax.ops.segment_sum(sm, col, num_segments=N)

    grid_n = N // N_BLOCK
    vector, node_attr = pl.pallas_call(
        _node_body,
        grid=(grid_n,),
        in_specs=[
            pl.BlockSpec((N_BLOCK, 32), lambda i: (i, 0)),
            pl.BlockSpec((N_BLOCK, 32), lambda i: (i, 0)),
            pl.BlockSpec((N_BLOCK, 3), lambda i: (i, 0)),
            pl.BlockSpec((N_BLOCK, 3), lambda i: (i, 0)),
            _full(nod_W0.shape), _full((1, 64)), _full(nod_W1.shape), _full((1, 64)),
            _full(nod_W2.shape), _full((1, 32)),
        ],
        out_specs=[
            pl.BlockSpec((N_BLOCK, 3), lambda i: (i, 0)),
            pl.BlockSpec((N_BLOCK, 32), lambda i: (i, 0)),
        ],
        out_shape=[
            jax.ShapeDtypeStruct((N, 3), jnp.float32),
            jax.ShapeDtypeStruct((N, 32), jnp.float32),
        ],
        compiler_params=pltpu.CompilerParams(
            dimension_semantics=("arbitrary",),
        ),
    )(agg, h, vecsum, x,
      nod_W0, b2(nod_b0), nod_W1, b2(nod_b1), nod_W2, b2(nod_b2))

    return vector, node_attr


# speedup vs baseline: 3.4832x; 1.3588x over previous
import functools

import jax
import jax.numpy as jnp
from jax import lax
from jax.experimental import pallas as pl
from jax.experimental.pallas import tpu as pltpu
from jax.experimental.pallas import tpu_sc as plsc

E_BLOCK = 3200
N_BLOCK = 2000
TD = 48
GCH = 1000
N_WORKERS = 32


def _silu(v):
    return v * jax.nn.sigmoid(v)


def _make_gather(E):
    per_w = E // N_WORKERS
    nch = per_w // GCH
    mesh = plsc.VectorSubcoreMesh(core_axis_name="c", subcore_axis_name="s")

    @functools.partial(
        pl.kernel,
        out_type=jax.ShapeDtypeStruct((E, 128), jnp.float32),
        mesh=mesh,
        scratch_types=[
            pltpu.VMEM((GCH,), jnp.int32),
            pltpu.VMEM((GCH,), jnp.int32),
            pltpu.VMEM((GCH, TD), jnp.float32),
            pltpu.VMEM((GCH, TD), jnp.float32),
            pltpu.VMEM((GCH, 16), jnp.float32),
            pltpu.SemaphoreType.DMA,
            pltpu.SemaphoreType.DMA,
            pltpu.SemaphoreType.DMA,
        ],
        compiler_params=pltpu.CompilerParams(use_tc_tiling_on_sc=False),
    )
    def gather_k(t_hbm, row_hbm, col_hbm, ea_hbm, out_hbm,
                 idxr_v, idxc_v, bufr_v, bufc_v, bufe_v, semr, semc, seme):
        wid = lax.axis_index("s") * 2 + lax.axis_index("c")
        base = wid * per_w

        def body(i, carry):
            off = base + i * GCH
            pltpu.sync_copy(row_hbm.at[pl.ds(off, GCH)], idxr_v)
            pltpu.sync_copy(col_hbm.at[pl.ds(off, GCH)], idxc_v)
            ce = pltpu.async_copy(ea_hbm.at[pl.ds(off, GCH)], bufe_v, seme)
            cr = pltpu.async_copy(t_hbm.at[idxr_v], bufr_v, semr)
            cc = pltpu.async_copy(t_hbm.at[idxc_v], bufc_v, semc)
            cr.wait()
            pltpu.sync_copy(bufr_v, out_hbm.at[pl.ds(off, GCH), pl.ds(0, TD)])
            cc.wait()
            pltpu.sync_copy(bufc_v, out_hbm.at[pl.ds(off, GCH), pl.ds(TD, TD)])
            ce.wait()
            pltpu.sync_copy(bufe_v, out_hbm.at[pl.ds(off, GCH), pl.ds(2 * TD, 16)])
            return carry

        lax.fori_loop(0, nch, body, 0)

    return gather_k


def _edge_body(in_ref, xmask_ref,
               w0_ref, b0_ref, w1_ref, b1_ref, w2_ref, b2_ref,
               vw0_ref, vb0_ref, vw1_ref, vb1_ref, vw2_ref, vb2_ref,
               sw0_ref, sb0_ref, sw1_ref, sb1_ref, sw2_ref, sb2_ref,
               out_ref):
    g = in_ref[...]
    xm = xmask_ref[...]
    n2 = jnp.sum(g * g * xm, axis=1, keepdims=True)
    inv = 1.0 / jnp.maximum(jnp.sqrt(n2), 1e-12)
    scale = xm * inv + (1.0 - xm)
    pre = (g * scale) @ w0_ref[...] + b0_ref[...]
    l1 = _silu(pre)
    l2 = _silu(l1 @ w1_ref[...] + b1_ref[...])
    msg = l2 @ w2_ref[...] + b2_ref[...]
    v = _silu(msg @ vw0_ref[...] + vb0_ref[...])
    v = _silu(v @ vw1_ref[...] + vb1_ref[...])
    vm = v @ vw2_ref[...] + vb2_ref[...]
    s = _silu(msg @ sw0_ref[...] + sb0_ref[...])
    s = _silu(s @ sw1_ref[...] + sb1_ref[...])
    sm = s @ sw2_ref[...] + sb2_ref[...]
    out_ref[...] = jnp.concatenate([sm, vm, jnp.zeros_like(vm[:, :1])], axis=1)


def _node_body(agg_ref, h_ref, vs_ref, x_ref,
               w0_ref, b0_ref, w1_ref, b1_ref, w2_ref, b2_ref,
               vec_ref, na_ref):
    a = _silu(agg_ref[...] @ w0_ref[...] + b0_ref[...])
    a = _silu(a @ w1_ref[...] + b1_ref[...])
    na_ref[...] = (a @ w2_ref[...] + b2_ref[...]) + h_ref[...]
    vec_ref[...] = x_ref[...] + vs_ref[...]


def _full(shape):
    return pl.BlockSpec(shape, lambda i: (0,) * len(shape))


def kernel(edge_index, edge_attr, x, h,
           msg_W0, msg_b0, msg_W1, msg_b1, msg_W2, msg_b2,
           vec_W0, vec_b0, vec_W1, vec_b1, vec_W2, vec_b2,
           sca_W0, sca_b0, sca_W1, sca_b1, sca_W2, sca_b2,
           nod_W0, nod_b0, nod_W1, nod_b1, nod_W2, nod_b2):
    E = edge_index.shape[1]
    N = x.shape[0]
    row = edge_index[0]
    col = edge_index[1]

    T = jnp.concatenate(
        [x, jnp.zeros((N, 5), jnp.float32), h, jnp.zeros((N, 8), jnp.float32)],
        axis=1)

    packed = _make_gather(E)(T, row, col, edge_attr)

    z = jnp.zeros((5, 64), jnp.float32)
    z8 = jnp.zeros((8, 64), jnp.float32)
    w0p = jnp.concatenate([
        msg_W0[0:3], z, msg_W0[6:38], z8,
        msg_W0[3:6], z, msg_W0[38:70], z8,
        msg_W0[70:86], jnp.zeros((16, 64), jnp.float32),
    ], axis=0)
    xmask = jnp.zeros((1, 128), jnp.float32)
    xmask = xmask.at[0, 0:3].set(1.0).at[0, 48:51].set(1.0)
    b2 = lambda b: b.reshape(1, -1)

    grid_e = E // E_BLOCK
    msgs = pl.pallas_call(
        _edge_body,
        grid=(grid_e,),
        in_specs=[
            pl.BlockSpec((E_BLOCK, 128), lambda i: (i, 0)),
            _full((1, 128)),
            _full((128, 64)), _full((1, 64)),
            _full(msg_W1.shape), _full((1, 64)), _full(msg_W2.shape), _full((1, 64)),
            _full(vec_W0.shape), _full((1, 64)), _full(vec_W1.shape), _full((1, 64)),
            _full(vec_W2.shape), _full((1, 3)),
            _full(sca_W0.shape), _full((1, 64)), _full(sca_W1.shape), _full((1, 64)),
            _full(sca_W2.shape), _full((1, 32)),
        ],
        out_specs=pl.BlockSpec((E_BLOCK, 36), lambda i: (i, 0)),
        out_shape=jax.ShapeDtypeStruct((E, 36), jnp.float32),
        compiler_params=pltpu.CompilerParams(
            dimension_semantics=("arbitrary",),
        ),
    )(packed, xmask,
      w0p, b2(msg_b0), msg_W1, b2(msg_b1), msg_W2, b2(msg_b2),
      vec_W0, b2(vec_b0), vec_W1, b2(vec_b1), vec_W2, b2(vec_b2),
      sca_W0, b2(sca_b0), sca_W1, b2(sca_b1), sca_W2, b2(sca_b2))

    summed = jax.ops.segment_sum(msgs, col, num_segments=N)
    agg = summed[:, 0:32]
    vecsum = summed[:, 32:35]

    grid_n = N // N_BLOCK
    vector, node_attr = pl.pallas_call(
        _node_body,
        grid=(grid_n,),
        in_specs=[
            pl.BlockSpec((N_BLOCK, 32), lambda i: (i, 0)),
            pl.BlockSpec((N_BLOCK, 32), lambda i: (i, 0)),
            pl.BlockSpec((N_BLOCK, 3), lambda i: (i, 0)),
            pl.BlockSpec((N_BLOCK, 3), lambda i: (i, 0)),
            _full(nod_W0.shape), _full((1, 64)), _full(nod_W1.shape), _full((1, 64)),
            _full(nod_W2.shape), _full((1, 32)),
        ],
        out_specs=[
            pl.BlockSpec((N_BLOCK, 3), lambda i: (i, 0)),
            pl.BlockSpec((N_BLOCK, 32), lambda i: (i, 0)),
        ],
        out_shape=[
            jax.ShapeDtypeStruct((N, 3), jnp.float32),
            jax.ShapeDtypeStruct((N, 32), jnp.float32),
        ],
        compiler_params=pltpu.CompilerParams(
            dimension_semantics=("arbitrary",),
        ),
    )(agg, h, vecsum, x,
      nod_W0, b2(nod_b0), nod_W1, b2(nod_b1), nod_W2, b2(nod_b2))

    return vector, node_attr
